# Initial kernel scaffold; baseline (speedup 1.0000x reference)
import jax
import jax.numpy as jnp
from jax.experimental import pallas as pl


def kernel(h, forward_indices, W, b):
    raise NotImplementedError("write your pallas kernel here")


# speedup vs baseline: 35.4936x; 35.4936x over previous
import functools

import jax
import jax.numpy as jnp
from jax import lax
from jax.experimental import pallas as pl
from jax.experimental.pallas import tpu as pltpu
from jax.experimental.pallas import tpu_sc as plsc

N_NODES = 10000
D_FEAT = 128
N_PAIRS = 320000

_info = plsc.get_sparse_core_info()
_NC, _NS, _L = _info.num_cores, _info.num_subcores, _info.num_lanes
_NW = _NC * _NS
_BPW = N_PAIRS // _NW


def _matvec_body(h_ref, w2_ref, b_ref, o_ref):
    w = w2_ref[0:1, :] + w2_ref[1:2, :]
    o_ref[...] = jnp.sum(h_ref[...] * w, axis=1, keepdims=True) + b_ref[...]


_matvec = pl.pallas_call(
    _matvec_body,
    out_shape=jax.ShapeDtypeStruct((N_NODES, 1), jnp.float32),
)


_mesh = plsc.VectorSubcoreMesh(core_axis_name="c", subcore_axis_name="s")


@functools.partial(
    pl.kernel,
    mesh=_mesh,
    out_type=jax.ShapeDtypeStruct((N_PAIRS,), jnp.float32),
    compiler_params=pltpu.CompilerParams(needs_layout_passes=False),
    scratch_types=[
        pltpu.VMEM((N_NODES,), jnp.float32),
        pltpu.VMEM((_BPW,), jnp.int32),
        pltpu.VMEM((_BPW,), jnp.int32),
        pltpu.VMEM((_BPW,), jnp.float32),
    ],
)
def _sc_pool(s_hbm, idx_hbm, out_hbm, s_v, i0_v, i1_v, o_v):
    wid = lax.axis_index("s") * _NC + lax.axis_index("c")
    base = wid * _BPW
    pltpu.sync_copy(s_hbm, s_v)
    pltpu.sync_copy(idx_hbm.at[pl.ds(base, _BPW)], i0_v)
    pltpu.sync_copy(idx_hbm.at[pl.ds(N_PAIRS + base, _BPW)], i1_v)

    def body(t, carry):
        off = t * _L
        v0 = plsc.load_gather(s_v, [i0_v[pl.ds(off, _L)]])
        v1 = plsc.load_gather(s_v, [i1_v[pl.ds(off, _L)]])
        o_v[pl.ds(off, _L)] = v0 + v1
        return carry

    lax.fori_loop(0, _BPW // _L, body, 0)
    pltpu.sync_copy(o_v, out_hbm.at[pl.ds(base, _BPW)])


def kernel(h, forward_indices, W, b):
    w2 = W[: 2 * D_FEAT, 0].reshape(2, D_FEAT)
    s = _matvec(h, w2, b.reshape(1, 1)).reshape(N_NODES)
    out = _sc_pool(s, forward_indices.reshape(-1))
    return out.reshape(N_PAIRS, 1)


# speedup vs baseline: 40.6215x; 1.1445x over previous
import functools

import jax
import jax.numpy as jnp
from jax import lax
from jax.experimental import pallas as pl
from jax.experimental.pallas import tpu as pltpu
from jax.experimental.pallas import tpu_sc as plsc

N_NODES = 10000
D_FEAT = 128
N_PAIRS = 320000

_info = plsc.get_sparse_core_info()
_NC, _NS, _L = _info.num_cores, _info.num_subcores, _info.num_lanes
_NW = _NC * _NS
_BPW = N_PAIRS // _NW


def _matvec_body(h_ref, w2_ref, b_ref, o_ref):
    w = w2_ref[0:1, :] + w2_ref[1:2, :]
    o_ref[...] = jnp.sum(h_ref[...] * w, axis=1) + b_ref[0]


_matvec = pl.pallas_call(
    _matvec_body,
    out_shape=jax.ShapeDtypeStruct((N_NODES,), jnp.float32),
)


_mesh = plsc.VectorSubcoreMesh(core_axis_name="c", subcore_axis_name="s")


@functools.partial(
    pl.kernel,
    mesh=_mesh,
    out_type=jax.ShapeDtypeStruct((N_PAIRS,), jnp.float32),
    compiler_params=pltpu.CompilerParams(needs_layout_passes=False),
    scratch_types=[
        pltpu.VMEM((N_NODES,), jnp.float32),
        pltpu.VMEM((_BPW,), jnp.int32),
        pltpu.VMEM((_BPW,), jnp.int32),
        pltpu.VMEM((_BPW,), jnp.float32),
    ],
)
def _sc_pool(s_hbm, idx_hbm, out_hbm, s_v, i0_v, i1_v, o_v):
    wid = lax.axis_index("s") * _NC + lax.axis_index("c")
    base = wid * _BPW
    pltpu.sync_copy(s_hbm, s_v)
    pltpu.sync_copy(idx_hbm.at[pl.ds(base, _BPW)], i0_v)
    pltpu.sync_copy(idx_hbm.at[pl.ds(N_PAIRS + base, _BPW)], i1_v)

    @plsc.parallel_loop(0, _BPW, step=_L, unroll=8)
    def _(off):
        v0 = plsc.load_gather(s_v, [i0_v[pl.ds(off, _L)]])
        v1 = plsc.load_gather(s_v, [i1_v[pl.ds(off, _L)]])
        o_v[pl.ds(off, _L)] = v0 + v1
    pltpu.sync_copy(o_v, out_hbm.at[pl.ds(base, _BPW)])


def kernel(h, forward_indices, W, b):
    w2 = W[: 2 * D_FEAT, 0].reshape(2, D_FEAT)
    s = _matvec(h, w2, b)
    out = _sc_pool(s, forward_indices.reshape(-1))
    return out.reshape(N_PAIRS, 1)


# speedup vs baseline: 46.5672x; 1.1464x over previous
import functools

import jax
import jax.numpy as jnp
from jax import lax
from jax.experimental import pallas as pl
from jax.experimental.pallas import tpu as pltpu
from jax.experimental.pallas import tpu_sc as plsc

N_NODES = 10000
D_FEAT = 128
N_PAIRS = 320000

_info = plsc.get_sparse_core_info()
_NC, _NS, _L = _info.num_cores, _info.num_subcores, _info.num_lanes
_NW = _NC * _NS
_BPW = N_PAIRS // _NW
_CHUNK = 10240


def _matvec_body(h_ref, w2_ref, b_ref, o_ref):
    w = w2_ref[0:1, :] + w2_ref[1:2, :]
    o_ref[...] = jnp.sum(h_ref[...] * w, axis=1) + b_ref[0]


_matvec = pl.pallas_call(
    _matvec_body,
    out_shape=jax.ShapeDtypeStruct((N_NODES,), jnp.float32),
)


_mesh = plsc.VectorSubcoreMesh(core_axis_name="c", subcore_axis_name="s")


@functools.partial(
    pl.kernel,
    mesh=_mesh,
    out_type=jax.ShapeDtypeStruct((N_PAIRS,), jnp.float32),
    compiler_params=pltpu.CompilerParams(needs_layout_passes=False),
    scratch_types=[
        pltpu.VMEM((N_NODES,), jnp.float32),
        pltpu.VMEM((2, _CHUNK), jnp.int32),
        pltpu.VMEM((_BPW,), jnp.float32),
    ],
)
def _sc_pool(s_hbm, idx_hbm, out_hbm, s_v, i01_v, o_v):
    wid = lax.axis_index("s") * _NC + lax.axis_index("c")
    base = wid * _BPW
    base_al = pl.multiple_of(
        jnp.minimum((base // 128) * 128, N_PAIRS - _CHUNK), 128
    )
    extra = base - base_al
    pltpu.sync_copy(s_hbm, s_v)
    pltpu.sync_copy(idx_hbm.at[:, pl.ds(base_al, _CHUNK)], i01_v)

    @plsc.parallel_loop(0, _BPW, step=_L, unroll=8)
    def _(off):
        v0 = plsc.load_gather(s_v, [i01_v[0, pl.ds(extra + off, _L)]])
        v1 = plsc.load_gather(s_v, [i01_v[1, pl.ds(extra + off, _L)]])
        o_v[pl.ds(off, _L)] = v0 + v1
    pltpu.sync_copy(o_v, out_hbm.at[pl.ds(base, _BPW)])


def kernel(h, forward_indices, W, b):
    w2 = W[: 2 * D_FEAT, 0].reshape(2, D_FEAT)
    s = _matvec(h, w2, b)
    return _sc_pool(s, forward_indices).reshape(N_PAIRS, 1)


# speedup vs baseline: 46.6388x; 1.0015x over previous
import functools

import jax
import jax.numpy as jnp
from jax import lax
from jax.experimental import pallas as pl
from jax.experimental.pallas import tpu as pltpu
from jax.experimental.pallas import tpu_sc as plsc

N_NODES = 10000
D_FEAT = 128
N_PAIRS = 320000

_info = plsc.get_sparse_core_info()
_NC, _NS, _L = _info.num_cores, _info.num_subcores, _info.num_lanes
_NW = _NC * _NS
_BPW = N_PAIRS // _NW
_CHUNK = 10240


def _matvec_body(h_ref, w2_ref, b_ref, o_ref):
    w = w2_ref[0:1, :] + w2_ref[1:2, :]
    s = jax.lax.dot_general(
        w,
        h_ref[...],
        dimension_numbers=(((1,), (1,)), ((), ())),
        preferred_element_type=jnp.float32,
        precision=jax.lax.Precision.HIGHEST,
    )
    o_ref[...] = s + b_ref[0]


_matvec = pl.pallas_call(
    _matvec_body,
    out_shape=jax.ShapeDtypeStruct((1, N_NODES), jnp.float32),
)


_mesh = plsc.VectorSubcoreMesh(core_axis_name="c", subcore_axis_name="s")


@functools.partial(
    pl.kernel,
    mesh=_mesh,
    out_type=jax.ShapeDtypeStruct((N_PAIRS,), jnp.float32),
    compiler_params=pltpu.CompilerParams(needs_layout_passes=False),
    scratch_types=[
        pltpu.VMEM((N_NODES,), jnp.float32),
        pltpu.VMEM((2, _CHUNK), jnp.int32),
        pltpu.VMEM((_BPW,), jnp.float32),
    ],
)
def _sc_pool(s_hbm, idx_hbm, out_hbm, s_v, i01_v, o_v):
    wid = lax.axis_index("s") * _NC + lax.axis_index("c")
    base = wid * _BPW
    base_al = pl.multiple_of(
        jnp.minimum((base // 128) * 128, N_PAIRS - _CHUNK), 128
    )
    extra = base - base_al
    pltpu.sync_copy(s_hbm.at[0], s_v)
    pltpu.sync_copy(idx_hbm.at[:, pl.ds(base_al, _CHUNK)], i01_v)

    @plsc.parallel_loop(0, _BPW, step=_L, unroll=8)
    def _(off):
        v0 = plsc.load_gather(s_v, [i01_v[0, pl.ds(extra + off, _L)]])
        v1 = plsc.load_gather(s_v, [i01_v[1, pl.ds(extra + off, _L)]])
        o_v[pl.ds(off, _L)] = v0 + v1
    pltpu.sync_copy(o_v, out_hbm.at[pl.ds(base, _BPW)])


def kernel(h, forward_indices, W, b):
    w2 = W[: 2 * D_FEAT, 0].reshape(2, D_FEAT)
    s = _matvec(h, w2, b)
    return _sc_pool(s, forward_indices).reshape(N_PAIRS, 1)
